# baseline (speedup 1.0000x reference)
import jax
import jax.numpy as jnp
from jax.experimental import pallas as pl

UNITS = 64
E = 16
D_EXP = 128
FEAT = 1024
H = 4
TOPK = 4
BN_EPS = 1e-5
TB = 256


def _moe_block(feat_ref, x_ref, wexp_ref, bexp_ref, gamma_ref, beta_ref,
               ebias_ref, wgm_ref, bg_ref, gbias_ref, gw_ref,
               out_ref, eo_ref):
    inv_bn = 1.0 / jnp.sqrt(1.0 + BN_EPS)
    nt = feat_ref.shape[0]

    bexpT = bexp_ref[...].T
    gammaT = gamma_ref[...].T
    betaT = beta_ref[...].T
    ebiasT = ebias_ref[...].T

    gT = jax.lax.dot_general(
        wgm_ref[...], feat_ref[...], (((1,), (1,)), ((), ())),
        preferred_element_type=jnp.float32)

    gw = gw_ref[...]
    m = jnp.max(gw, axis=1, keepdims=True)
    egw = jnp.exp((gw - m) / 0.01)
    ngwT = (egw / jnp.sum(egw, axis=1, keepdims=True)).T

    siota = jax.lax.broadcasted_iota(jnp.int32, (E, nt), 0)
    wdense = []
    for i in range(H):
        gb_i = (bg_ref[i:i + 1, :] + gbias_ref[i:i + 1, :]).T
        vals = jnp.maximum(gT[i * E:(i + 1) * E, :] + gb_i, 0.0) * ngwT
        topv, onehots = [], []
        for _ in range(TOPK):
            mk = jnp.max(vals, axis=0, keepdims=True)
            idx = jnp.min(jnp.where(vals == mk, siota, E),
                          axis=0, keepdims=True)
            sel = siota == idx
            topv.append(mk)
            onehots.append(sel)
            vals = jnp.where(sel, -1e30, vals)
        exps = [jnp.exp((v - topv[0]) / 0.01) for v in topv]
        denom = exps[0] + exps[1] + exps[2] + exps[3]
        wd = jnp.zeros((E, nt), dtype=jnp.float32)
        for k in range(TOPK):
            wd = wd + jnp.where(onehots[k], exps[k] / denom, 0.0)
        wdense.append(wd)

    heads = [jnp.zeros((UNITS, nt), dtype=jnp.float32) for _ in range(H)]
    oT_list = []
    for e in range(E):
        x_e = x_ref[:, e * D_EXP:(e + 1) * D_EXP]
        oT = jax.lax.dot_general(
            wexp_ref[e], x_e, (((1,), (1,)), ((), ())),
            preferred_element_type=jnp.float32)
        oT = oT + bexpT[:, e:e + 1]
        oT = oT * (inv_bn * gammaT[:, e:e + 1]) + betaT[:, e:e + 1]
        oT = jnp.maximum(oT, 0.0)
        oT = oT + ebiasT[:, e:e + 1]
        oT = jnp.where(oT >= 0.0, oT, 0.01 * oT)
        oT_list.append(oT)
        for i in range(H):
            heads[i] = heads[i] + wdense[i][e:e + 1, :] * oT
    eo_ref[...] = jnp.stack(oT_list, axis=1).reshape(E * UNITS, nt)
    for i in range(H):
        out_ref[:, i * UNITS:(i + 1) * UNITS] = heads[i].T


def kernel(feature_input, inputs, Wexp, bexp, bn_gamma, bn_beta, extra_bias,
           Wg, bg, gbias, global_weights):
    B = feature_input.shape[0]
    wgm = Wg.reshape(H * E, FEAT)
    gw = global_weights.reshape(1, E)

    grid = (B // TB,)
    out, eoT = pl.pallas_call(
        _moe_block,
        grid=grid,
        in_specs=[
            pl.BlockSpec((TB, FEAT), lambda i: (i, 0)),
            pl.BlockSpec((TB, E * D_EXP), lambda i: (i, 0)),
            pl.BlockSpec((E, UNITS, D_EXP), lambda i: (0, 0, 0)),
            pl.BlockSpec((E, UNITS), lambda i: (0, 0)),
            pl.BlockSpec((E, UNITS), lambda i: (0, 0)),
            pl.BlockSpec((E, UNITS), lambda i: (0, 0)),
            pl.BlockSpec((E, UNITS), lambda i: (0, 0)),
            pl.BlockSpec((H * E, FEAT), lambda i: (0, 0)),
            pl.BlockSpec((H, E), lambda i: (0, 0)),
            pl.BlockSpec((H, E), lambda i: (0, 0)),
            pl.BlockSpec((1, E), lambda i: (0, 0)),
        ],
        out_specs=[
            pl.BlockSpec((TB, H * UNITS), lambda i: (i, 0)),
            pl.BlockSpec((UNITS * E, TB), lambda i: (0, i)),
        ],
        out_shape=[
            jax.ShapeDtypeStruct((B, H * UNITS), jnp.float32),
            jax.ShapeDtypeStruct((UNITS * E, B), jnp.float32),
        ],
    )(feature_input, inputs, Wexp, bexp, bn_gamma, bn_beta, extra_bias,
      wgm, bg, gbias, gw)
    eo = eoT.reshape(UNITS, E, B).transpose(2, 0, 1)
    return (out, eo)


# speedup vs baseline: 1.0448x; 1.0448x over previous
import jax
import jax.numpy as jnp
from jax.experimental import pallas as pl

UNITS = 64
E = 16
D_EXP = 128
FEAT = 1024
H = 4
TOPK = 4
BN_EPS = 1e-5
TB = 512


def _moe_block(feat_ref, x_ref, wexp_ref, bexp_ref, gamma_ref, beta_ref,
               ebias_ref, wgm_ref, bg_ref, gbias_ref, gw_ref,
               out_ref, eo_ref):
    inv_bn = 1.0 / jnp.sqrt(1.0 + BN_EPS)
    nt = feat_ref.shape[0]

    bexpT = bexp_ref[...].T
    gammaT = gamma_ref[...].T
    betaT = beta_ref[...].T
    ebiasT = ebias_ref[...].T

    gT = jax.lax.dot_general(
        wgm_ref[...], feat_ref[...], (((1,), (1,)), ((), ())),
        preferred_element_type=jnp.float32)

    gw = gw_ref[...]
    m = jnp.max(gw, axis=1, keepdims=True)
    egw = jnp.exp((gw - m) / 0.01)
    ngwT = (egw / jnp.sum(egw, axis=1, keepdims=True)).T

    siota = jax.lax.broadcasted_iota(jnp.int32, (E, nt), 0)
    wdense = []
    for i in range(H):
        gb_i = (bg_ref[i:i + 1, :] + gbias_ref[i:i + 1, :]).T
        vals = jnp.maximum(gT[i * E:(i + 1) * E, :] + gb_i, 0.0) * ngwT
        topv, onehots = [], []
        for _ in range(TOPK):
            mk = jnp.max(vals, axis=0, keepdims=True)
            idx = jnp.min(jnp.where(vals == mk, siota, E),
                          axis=0, keepdims=True)
            sel = siota == idx
            topv.append(mk)
            onehots.append(sel)
            vals = jnp.where(sel, -1e30, vals)
        exps = [jnp.exp((v - topv[0]) / 0.01) for v in topv]
        denom = exps[0] + exps[1] + exps[2] + exps[3]
        wd = jnp.zeros((E, nt), dtype=jnp.float32)
        for k in range(TOPK):
            wd = wd + jnp.where(onehots[k], exps[k] / denom, 0.0)
        wdense.append(wd)

    heads = [jnp.zeros((UNITS, nt), dtype=jnp.float32) for _ in range(H)]
    oT_list = []
    for e in range(E):
        x_e = x_ref[:, e * D_EXP:(e + 1) * D_EXP]
        oT = jax.lax.dot_general(
            wexp_ref[e], x_e, (((1,), (1,)), ((), ())),
            preferred_element_type=jnp.float32)
        oT = oT + bexpT[:, e:e + 1]
        oT = oT * (inv_bn * gammaT[:, e:e + 1]) + betaT[:, e:e + 1]
        oT = jnp.maximum(oT, 0.0)
        oT = oT + ebiasT[:, e:e + 1]
        oT = jnp.where(oT >= 0.0, oT, 0.01 * oT)
        oT_list.append(oT)
        for i in range(H):
            heads[i] = heads[i] + wdense[i][e:e + 1, :] * oT
    eo_ref[...] = jnp.stack(oT_list, axis=1).reshape(E * UNITS, nt)
    for i in range(H):
        out_ref[:, i * UNITS:(i + 1) * UNITS] = heads[i].T


def kernel(feature_input, inputs, Wexp, bexp, bn_gamma, bn_beta, extra_bias,
           Wg, bg, gbias, global_weights):
    B = feature_input.shape[0]
    wgm = Wg.reshape(H * E, FEAT)
    gw = global_weights.reshape(1, E)

    grid = (B // TB,)
    out, eoT = pl.pallas_call(
        _moe_block,
        grid=grid,
        in_specs=[
            pl.BlockSpec((TB, FEAT), lambda i: (i, 0)),
            pl.BlockSpec((TB, E * D_EXP), lambda i: (i, 0)),
            pl.BlockSpec((E, UNITS, D_EXP), lambda i: (0, 0, 0)),
            pl.BlockSpec((E, UNITS), lambda i: (0, 0)),
            pl.BlockSpec((E, UNITS), lambda i: (0, 0)),
            pl.BlockSpec((E, UNITS), lambda i: (0, 0)),
            pl.BlockSpec((E, UNITS), lambda i: (0, 0)),
            pl.BlockSpec((H * E, FEAT), lambda i: (0, 0)),
            pl.BlockSpec((H, E), lambda i: (0, 0)),
            pl.BlockSpec((H, E), lambda i: (0, 0)),
            pl.BlockSpec((1, E), lambda i: (0, 0)),
        ],
        out_specs=[
            pl.BlockSpec((TB, H * UNITS), lambda i: (i, 0)),
            pl.BlockSpec((UNITS * E, TB), lambda i: (0, i)),
        ],
        out_shape=[
            jax.ShapeDtypeStruct((B, H * UNITS), jnp.float32),
            jax.ShapeDtypeStruct((UNITS * E, B), jnp.float32),
        ],
    )(feature_input, inputs, Wexp, bexp, bn_gamma, bn_beta, extra_bias,
      wgm, bg, gbias, gw)
    eo = eoT.reshape(UNITS, E, B).transpose(2, 0, 1)
    return (out, eo)
